# Initial kernel scaffold; baseline (speedup 1.0000x reference)
import jax
import jax.numpy as jnp
from jax.experimental import pallas as pl


def kernel(x, adj, Wp0, Wg0, Wl0, Wp1, Wg1, Wl1, Wp2, Wg2, Wl2):
    raise NotImplementedError("write your pallas kernel here")


# speedup vs baseline: 1.1089x; 1.1089x over previous
import functools

import jax
import jax.numpy as jnp
from jax.experimental import pallas as pl
from jax.experimental.pallas import tpu as pltpu

N = 2048
BM = 256

PREC_HI = jax.lax.Precision.DEFAULT
PREC_LO = jax.lax.Precision.DEFAULT


def _leaky(x):
    return jnp.where(x >= 0, x, 0.01 * x)


def _prep_body(h_ref, wp_ref, wg_ref, wl_ref, q_ref, v_ref, u_ref):
    h = h_ref[...]
    q_ref[...] = jnp.dot(h, wp_ref[...], precision=PREC_HI,
                         preferred_element_type=jnp.float32)
    v_ref[...] = jnp.dot(h, wg_ref[...], precision=PREC_LO,
                         preferred_element_type=jnp.float32)
    u_ref[...] = jnp.dot(h, wl_ref[...], precision=PREC_LO,
                         preferred_element_type=jnp.float32)


def _main_body(q_ref, k_ref, v_ref, u_ref, adj_ref, o_ref, *, act):
    logits = jax.lax.dot_general(
        q_ref[...], k_ref[...], (((1,), (1,)), ((), ())),
        precision=PREC_HI, preferred_element_type=jnp.float32)
    m = jnp.max(logits, axis=1, keepdims=True)
    e = jnp.exp(logits - m)
    s = jnp.sum(e, axis=1, keepdims=True)
    og = jnp.dot(e, v_ref[...], precision=PREC_LO,
                 preferred_element_type=jnp.float32) / s
    ol = jnp.dot(adj_ref[...], u_ref[...], precision=PREC_LO,
                 preferred_element_type=jnp.float32)
    if act:
        og = _leaky(og)
        ol = _leaky(ol)
    o_ref[...] = og + ol


def _layer(h, adj, Wp, Wg, Wl, act):
    cin = h.shape[1]
    cout = Wg.shape[1]
    f32 = jnp.float32

    q, v, u = pl.pallas_call(
        _prep_body,
        grid=(N // BM,),
        in_specs=[
            pl.BlockSpec((BM, cin), lambda i: (i, 0)),
            pl.BlockSpec((cin, cin), lambda i: (0, 0)),
            pl.BlockSpec((cin, cout), lambda i: (0, 0)),
            pl.BlockSpec((cin, cout), lambda i: (0, 0)),
        ],
        out_specs=[
            pl.BlockSpec((BM, cin), lambda i: (i, 0)),
            pl.BlockSpec((BM, cout), lambda i: (i, 0)),
            pl.BlockSpec((BM, cout), lambda i: (i, 0)),
        ],
        out_shape=[
            jax.ShapeDtypeStruct((N, cin), f32),
            jax.ShapeDtypeStruct((N, cout), f32),
            jax.ShapeDtypeStruct((N, cout), f32),
        ],
    )(h, Wp, Wg, Wl)

    out = pl.pallas_call(
        functools.partial(_main_body, act=act),
        grid=(N // BM,),
        in_specs=[
            pl.BlockSpec((BM, cin), lambda i: (i, 0)),
            pl.BlockSpec((N, cin), lambda i: (0, 0)),
            pl.BlockSpec((N, cout), lambda i: (0, 0)),
            pl.BlockSpec((N, cout), lambda i: (0, 0)),
            pl.BlockSpec((BM, N), lambda i: (i, 0)),
        ],
        out_specs=pl.BlockSpec((BM, cout), lambda i: (i, 0)),
        out_shape=jax.ShapeDtypeStruct((N, cout), f32),
    )(q, h, v, u, adj)
    return out


def kernel(x, adj, Wp0, Wg0, Wl0, Wp1, Wg1, Wl1, Wp2, Wg2, Wl2):
    h = _layer(x, adj, Wp0, Wg0, Wl0, act=True)
    h = _layer(h, adj, Wp1, Wg1, Wl1, act=True)
    return _layer(h, adj, Wp2, Wg2, Wl2, act=False)


# speedup vs baseline: 1.5532x; 1.4007x over previous
import functools

import jax
import jax.numpy as jnp
from jax.experimental import pallas as pl
from jax.experimental.pallas import tpu as pltpu

N = 2048
BM = 256

PREC = jax.lax.Precision.DEFAULT


def _leaky(x):
    return jnp.where(x >= 0, x, 0.01 * x)


def _layer_body(h_ref, wp_ref, wg_ref, wl_ref, adj_ref, o_ref,
                q_scr, v_scr, u_scr, *, act):
    i = pl.program_id(0)

    @pl.when(i == 0)
    def _prep():
        h = h_ref[...]
        q_scr[...] = jnp.dot(h, wp_ref[...], precision=PREC,
                             preferred_element_type=jnp.float32)
        v_scr[...] = jnp.dot(h, wg_ref[...], precision=PREC,
                             preferred_element_type=jnp.float32)
        u_scr[...] = jnp.dot(h, wl_ref[...], precision=PREC,
                             preferred_element_type=jnp.float32)

    q_i = q_scr[pl.ds(i * BM, BM), :]
    logits = jax.lax.dot_general(
        q_i, h_ref[...], (((1,), (1,)), ((), ())),
        precision=PREC, preferred_element_type=jnp.float32)
    m = jnp.max(logits, axis=1, keepdims=True)
    e = jnp.exp(logits - m)
    s = jnp.sum(e, axis=1, keepdims=True)
    og = jnp.dot(e, v_scr[...], precision=PREC,
                 preferred_element_type=jnp.float32) / s
    ol = jnp.dot(adj_ref[...], u_scr[...], precision=PREC,
                 preferred_element_type=jnp.float32)
    if act:
        og = _leaky(og)
        ol = _leaky(ol)
    o_ref[...] = og + ol


def _layer(h, adj, Wp, Wg, Wl, act):
    cin = h.shape[1]
    cout = Wg.shape[1]
    f32 = jnp.float32
    return pl.pallas_call(
        functools.partial(_layer_body, act=act),
        grid=(N // BM,),
        in_specs=[
            pl.BlockSpec((N, cin), lambda i: (0, 0)),
            pl.BlockSpec((cin, cin), lambda i: (0, 0)),
            pl.BlockSpec((cin, cout), lambda i: (0, 0)),
            pl.BlockSpec((cin, cout), lambda i: (0, 0)),
            pl.BlockSpec((BM, N), lambda i: (i, 0)),
        ],
        out_specs=pl.BlockSpec((BM, cout), lambda i: (i, 0)),
        out_shape=jax.ShapeDtypeStruct((N, cout), f32),
        scratch_shapes=[
            pltpu.VMEM((N, cin), f32),
            pltpu.VMEM((N, cout), f32),
            pltpu.VMEM((N, cout), f32),
        ],
    )(h, Wp, Wg, Wl, adj)


def kernel(x, adj, Wp0, Wg0, Wl0, Wp1, Wg1, Wl1, Wp2, Wg2, Wl2):
    h = _layer(x, adj, Wp0, Wg0, Wl0, act=True)
    h = _layer(h, adj, Wp1, Wg1, Wl1, act=True)
    return _layer(h, adj, Wp2, Wg2, Wl2, act=False)


# speedup vs baseline: 1.6615x; 1.0698x over previous
import jax
import jax.numpy as jnp
from jax.experimental import pallas as pl
from jax.experimental.pallas import tpu as pltpu

N = 2048
BM = 256
NB = N // BM

PREC = jax.lax.Precision.DEFAULT


def _leaky(x):
    return jnp.where(x >= 0, x, 0.01 * x)


def _dot(a, b):
    return jnp.dot(a, b, precision=PREC, preferred_element_type=jnp.float32)


def _body(x_ref, wp0_ref, wg0_ref, wl0_ref, wp1_ref, wg1_ref, wl1_ref,
          wp2_ref, wg2_ref, wl2_ref, adj_ref, o_ref,
          h1_scr, h2_scr, q_scr, v_scr, u_scr):
    t = pl.program_id(0)
    layer = t // NB
    i = t % NB

    def phase(h_ref, wp_ref, wg_ref, wl_ref, cin, cout, out_ref, act):
        @pl.when(i == 0)
        def _prep():
            h = h_ref[...]
            q_scr[:, :cin] = _dot(h, wp_ref[...])
            v_scr[:, :cout] = _dot(h, wg_ref[...])
            u_scr[:, :cout] = _dot(h, wl_ref[...])

        q_i = q_scr[pl.ds(i * BM, BM), :cin]
        logits = jax.lax.dot_general(
            q_i, h_ref[...], (((1,), (1,)), ((), ())),
            precision=PREC, preferred_element_type=jnp.float32)
        m = jnp.max(logits, axis=1, keepdims=True)
        e = jnp.exp(logits - m)
        s = jnp.sum(e, axis=1, keepdims=True)
        og = _dot(e, v_scr[:, :cout]) / s
        ol = _dot(adj_ref[...], u_scr[:, :cout])
        if act:
            out = _leaky(og) + _leaky(ol)
        else:
            out = og + ol
        out_ref[pl.ds(i * BM, BM), :] = out

    @pl.when(layer == 0)
    def _l0():
        phase(x_ref, wp0_ref, wg0_ref, wl0_ref, 256, 256, h1_scr, True)

    @pl.when(layer == 1)
    def _l1():
        phase(h1_scr, wp1_ref, wg1_ref, wl1_ref, 256, 512, h2_scr, True)

    @pl.when(layer == 2)
    def _l2():
        @pl.when(i == 0)
        def _prep():
            h = h2_scr[...]
            q_scr[...] = _dot(h, wp2_ref[...])
            v_scr[:, :64] = _dot(h, wg2_ref[...])
            u_scr[:, :64] = _dot(h, wl2_ref[...])

        q_i = q_scr[pl.ds(i * BM, BM), :]
        logits = jax.lax.dot_general(
            q_i, h2_scr[...], (((1,), (1,)), ((), ())),
            precision=PREC, preferred_element_type=jnp.float32)
        m = jnp.max(logits, axis=1, keepdims=True)
        e = jnp.exp(logits - m)
        s = jnp.sum(e, axis=1, keepdims=True)
        og = _dot(e, v_scr[:, :64]) / s
        ol = _dot(adj_ref[...], u_scr[:, :64])
        o_ref[pl.ds(i * BM, BM), :] = og + ol


def kernel(x, adj, Wp0, Wg0, Wl0, Wp1, Wg1, Wl1, Wp2, Wg2, Wl2):
    f32 = jnp.float32
    full = lambda a, b: pl.BlockSpec(a, b)
    return pl.pallas_call(
        _body,
        grid=(3 * NB,),
        in_specs=[
            pl.BlockSpec((N, 256), lambda t: (0, 0)),
            pl.BlockSpec((256, 256), lambda t: (0, 0)),
            pl.BlockSpec((256, 256), lambda t: (0, 0)),
            pl.BlockSpec((256, 256), lambda t: (0, 0)),
            pl.BlockSpec((256, 256), lambda t: (0, 0)),
            pl.BlockSpec((256, 512), lambda t: (0, 0)),
            pl.BlockSpec((256, 512), lambda t: (0, 0)),
            pl.BlockSpec((512, 512), lambda t: (0, 0)),
            pl.BlockSpec((512, 64), lambda t: (0, 0)),
            pl.BlockSpec((512, 64), lambda t: (0, 0)),
            pl.BlockSpec((BM, N), lambda t: (t % NB, 0)),
        ],
        out_specs=pl.BlockSpec((N, 64), lambda t: (0, 0)),
        out_shape=jax.ShapeDtypeStruct((N, 64), f32),
        scratch_shapes=[
            pltpu.VMEM((N, 256), f32),
            pltpu.VMEM((N, 512), f32),
            pltpu.VMEM((N, 512), f32),
            pltpu.VMEM((N, 512), f32),
            pltpu.VMEM((N, 512), f32),
        ],
    )(x, Wp0, Wg0, Wl0, Wp1, Wg1, Wl1, Wp2, Wg2, Wl2, adj)
